# baseline (speedup 1.0000x reference)
import functools

import jax
import jax.numpy as jnp
from jax import lax
from jax.experimental import pallas as pl
from jax.experimental.pallas import tpu as pltpu

_THRESH = 0.7
_MIN_KEPT = 100000

_N, _C, _H, _W = 2, 150, 512, 512
_HB = 64
_STEPS = _N * (_H // _HB)

_NLOG_THRESH = float(-jnp.log(jnp.float32(_THRESH)))


def _pixel_stats(x, t):
    m = jnp.max(x, axis=0)
    tlo = t & 15
    thi = t >> 4
    mlo = [tlo == b for b in range(16)]
    s = jnp.zeros_like(m)
    xt = jnp.zeros_like(m)
    for a in range((_C + 15) // 16):
        nb = min(16, _C - a * 16)
        ya = None
        for b in range(nb):
            xc = x[a * 16 + b]
            s = s + jnp.exp(xc - m)
            sel = jnp.where(mlo[b], xc, 0.0)
            ya = sel if ya is None else ya + sel
        xt = xt + jnp.where(thi == a, ya, 0.0)
    return jnp.log(s) + m - xt


def _fused_body(pred_ref, tgt_ref, loss_ref, cnt_ref, acc_cnt_ref, acc_sum_ref):
    step = pl.program_id(0) * (_H // _HB) + pl.program_id(1)

    @pl.when(step == 0)
    def _init():
        acc_cnt_ref[...] = jnp.zeros_like(acc_cnt_ref)
        acc_sum_ref[...] = jnp.zeros_like(acc_sum_ref)

    nll = _pixel_stats(pred_ref[0], tgt_ref[0])
    mask = nll >= _NLOG_THRESH
    acc_cnt_ref[...] += mask.astype(jnp.float32)
    acc_sum_ref[...] += jnp.where(mask, nll, 0.0)

    @pl.when(step == _STEPS - 1)
    def _fin():
        cnt = jnp.sum(acc_cnt_ref[...])
        loss_ref[0, 0] = jnp.sum(acc_sum_ref[...]) / jnp.maximum(cnt, 1.0)
        cnt_ref[0, 0] = cnt


def _nll_body(pred_ref, tgt_ref, nll_ref):
    nll_ref[0] = _pixel_stats(pred_ref[0], tgt_ref[0])


def _select_body(nll_ref, out_ref):
    nll = nll_ref[...]
    p = jnp.exp(-nll)
    bits = lax.bitcast_convert_type(p, jnp.int32)
    k = jnp.int32(_MIN_KEPT)

    def step(i, prefix):
        b = 30 - i
        cand = prefix + (jnp.int32(1) << b)
        cnt = jnp.sum((bits < cand).astype(jnp.int32))
        return jnp.where(cnt >= k, prefix, cand)

    vbits = lax.fori_loop(0, 31, step, jnp.int32(0))
    thr = lax.bitcast_convert_type(vbits, jnp.float32)
    threshold = jnp.maximum(thr, jnp.float32(_THRESH))

    kept = p <= threshold
    cnt = jnp.sum(kept.astype(jnp.float32))
    ssum = jnp.sum(jnp.where(kept, nll, 0.0))
    out_ref[0, 0] = ssum / jnp.maximum(cnt, 1.0)


def _rare_select(pred, target):
    nll = pl.pallas_call(
        _nll_body,
        grid=(_N, _H // _HB),
        in_specs=[
            pl.BlockSpec((1, _C, _HB, _W), lambda n, h: (n, 0, h, 0)),
            pl.BlockSpec((1, _HB, _W), lambda n, h: (n, h, 0)),
        ],
        out_specs=pl.BlockSpec((1, _HB, _W), lambda n, h: (n, h, 0)),
        out_shape=jax.ShapeDtypeStruct((_N, _H, _W), jnp.float32),
        compiler_params=pltpu.CompilerParams(
            dimension_semantics=("arbitrary", "arbitrary"),
        ),
    )(pred, target)
    loss = pl.pallas_call(
        _select_body,
        in_specs=[pl.BlockSpec(memory_space=pltpu.MemorySpace.VMEM)],
        out_specs=pl.BlockSpec(memory_space=pltpu.MemorySpace.SMEM),
        out_shape=jax.ShapeDtypeStruct((1, 1), jnp.float32),
    )(nll)
    return loss[0, 0]


@jax.jit
def _ohem(pred, target):
    loss_fast, cnt = pl.pallas_call(
        _fused_body,
        grid=(_N, _H // _HB),
        in_specs=[
            pl.BlockSpec((1, _C, _HB, _W), lambda n, h: (n, 0, h, 0)),
            pl.BlockSpec((1, _HB, _W), lambda n, h: (n, h, 0)),
        ],
        out_specs=[
            pl.BlockSpec(memory_space=pltpu.MemorySpace.SMEM),
            pl.BlockSpec(memory_space=pltpu.MemorySpace.SMEM),
        ],
        out_shape=[
            jax.ShapeDtypeStruct((1, 1), jnp.float32),
            jax.ShapeDtypeStruct((1, 1), jnp.float32),
        ],
        scratch_shapes=[
            pltpu.VMEM((_HB, _W), jnp.float32),
            pltpu.VMEM((_HB, _W), jnp.float32),
        ],
        compiler_params=pltpu.CompilerParams(
            dimension_semantics=("arbitrary", "arbitrary"),
        ),
    )(pred, target)

    return lax.cond(
        cnt[0, 0] >= jnp.float32(_MIN_KEPT),
        lambda: loss_fast[0, 0],
        lambda: _rare_select(pred, target),
    )


def kernel(pred, target, epoch_i):
    return _ohem(pred, target)


# speedup vs baseline: 1.0067x; 1.0067x over previous
import functools

import jax
import jax.numpy as jnp
from jax import lax
from jax.experimental import pallas as pl
from jax.experimental.pallas import tpu as pltpu

_THRESH = 0.7
_MIN_KEPT = 100000

_N, _C, _H, _W = 2, 150, 512, 512
_HB = 64
_STEPS = _N * (_H // _HB)


def _fused_body(pred_ref, tgt_ref, p_ref, nll_ref, loss_ref, cnt_ref,
                acc_cnt_ref, acc_sum_ref):
    step = pl.program_id(0) * (_H // _HB) + pl.program_id(1)

    @pl.when(step == 0)
    def _init():
        acc_cnt_ref[...] = jnp.zeros_like(acc_cnt_ref)
        acc_sum_ref[...] = jnp.zeros_like(acc_sum_ref)

    x = pred_ref[0]
    t = tgt_ref[0]
    m = jnp.max(x, axis=0)
    tlo = t & 15
    thi = t >> 4
    mlo = [tlo == b for b in range(16)]
    s = jnp.zeros_like(m)
    xt = jnp.zeros_like(m)
    for a in range((_C + 15) // 16):
        nb = min(16, _C - a * 16)
        ya = None
        for b in range(nb):
            xc = x[a * 16 + b]
            s = s + jnp.exp(xc - m)
            sel = jnp.where(mlo[b], xc, 0.0)
            ya = sel if ya is None else ya + sel
        xt = xt + jnp.where(thi == a, ya, 0.0)
    p = jnp.exp(xt - m) / s
    nll = jnp.log(s) + m - xt
    p_ref[0] = p
    nll_ref[0] = nll

    mask = p <= _THRESH
    acc_cnt_ref[...] += mask.astype(jnp.float32)
    acc_sum_ref[...] += jnp.where(mask, nll, 0.0)

    @pl.when(step == _STEPS - 1)
    def _fin():
        cnt = jnp.sum(acc_cnt_ref[...])
        loss_ref[0, 0] = jnp.sum(acc_sum_ref[...]) / jnp.maximum(cnt, 1.0)
        cnt_ref[0, 0] = cnt


def _select_body(p_ref, nll_ref, out_ref):
    p = p_ref[...]
    bits = lax.bitcast_convert_type(p, jnp.int32)
    k = jnp.int32(_MIN_KEPT)

    def step(i, prefix):
        b = 30 - i
        cand = prefix + (jnp.int32(1) << b)
        cnt = jnp.sum((bits < cand).astype(jnp.int32))
        return jnp.where(cnt >= k, prefix, cand)

    vbits = lax.fori_loop(0, 31, step, jnp.int32(0))
    thr = lax.bitcast_convert_type(vbits, jnp.float32)
    threshold = jnp.maximum(thr, jnp.float32(_THRESH))

    kept = p <= threshold
    cnt = jnp.sum(kept.astype(jnp.float32))
    ssum = jnp.sum(jnp.where(kept, nll_ref[...], 0.0))
    out_ref[0, 0] = ssum / jnp.maximum(cnt, 1.0)


def _rare_select(p, nll):
    loss = pl.pallas_call(
        _select_body,
        in_specs=[
            pl.BlockSpec(memory_space=pltpu.MemorySpace.VMEM),
            pl.BlockSpec(memory_space=pltpu.MemorySpace.VMEM),
        ],
        out_specs=pl.BlockSpec(memory_space=pltpu.MemorySpace.SMEM),
        out_shape=jax.ShapeDtypeStruct((1, 1), jnp.float32),
    )(p, nll)
    return loss[0, 0]


@jax.jit
def _ohem(pred, target):
    p, nll, loss_fast, cnt = pl.pallas_call(
        _fused_body,
        grid=(_N, _H // _HB),
        in_specs=[
            pl.BlockSpec((1, _C, _HB, _W), lambda n, h: (n, 0, h, 0)),
            pl.BlockSpec((1, _HB, _W), lambda n, h: (n, h, 0)),
        ],
        out_specs=[
            pl.BlockSpec((1, _HB, _W), lambda n, h: (n, h, 0)),
            pl.BlockSpec((1, _HB, _W), lambda n, h: (n, h, 0)),
            pl.BlockSpec(memory_space=pltpu.MemorySpace.SMEM),
            pl.BlockSpec(memory_space=pltpu.MemorySpace.SMEM),
        ],
        out_shape=[
            jax.ShapeDtypeStruct((_N, _H, _W), jnp.float32),
            jax.ShapeDtypeStruct((_N, _H, _W), jnp.float32),
            jax.ShapeDtypeStruct((1, 1), jnp.float32),
            jax.ShapeDtypeStruct((1, 1), jnp.float32),
        ],
        scratch_shapes=[
            pltpu.VMEM((_HB, _W), jnp.float32),
            pltpu.VMEM((_HB, _W), jnp.float32),
        ],
        compiler_params=pltpu.CompilerParams(
            dimension_semantics=("arbitrary", "arbitrary"),
        ),
    )(pred, target)

    return lax.cond(
        cnt[0, 0] >= jnp.float32(_MIN_KEPT),
        lambda: loss_fast[0, 0],
        lambda: _rare_select(p, nll),
    )


def kernel(pred, target, epoch_i):
    return _ohem(pred, target)
